# baseline (speedup 1.0000x reference)
import functools

import jax
import jax.numpy as jnp
from jax import lax
from jax.experimental import pallas as pl
from jax.experimental.pallas import tpu as pltpu
from jax.experimental.pallas import tpu_sc as plsc

B = 4096
S = 50
D = 512
NTAB = 225
TAB_PAD = 256
SENTINEL = 224

BLK = 256
NW = 32
RPW = B // NW
SPAD = 64


def _tc_body(tab_ref, sc_ref, oy_ref, d0, d1, d2, d3, d4, hr_ref,
             lng, lnb, wst, sb, oyat, dtab, hwt, hb, wt, ib, out_ref):
    i = pl.program_id(0)

    @pl.when(i == 0)
    def _():
        out_ref[...] = tab_ref[...]

    @pl.when(i > 0)
    def _():
        x = sc_ref[...]
        mu = jnp.mean(x, axis=-1, keepdims=True)
        xc = x - mu
        var = jnp.mean(xc * xc, axis=-1, keepdims=True)
        xn = xc * lax.rsqrt(var + 1e-5) * lng[...] + lnb[...]
        s_emb = jnp.dot(xn, wst[...], preferred_element_type=jnp.float32) + sb[...]

        oh = (oy_ref[...] == lax.broadcasted_iota(jnp.int32, (BLK, 4), 1))
        oya_emb = jnp.dot(oh.astype(jnp.float32), oyat[...],
                          preferred_element_type=jnp.float32)

        h_emb = jnp.dot(hr_ref[...], hwt[...],
                        preferred_element_type=jnp.float32) + hb[...]

        acc = jnp.dot(s_emb, wt[0:32, :], preferred_element_type=jnp.float32)
        acc += jnp.dot(oya_emb, wt[32:48, :], preferred_element_type=jnp.float32)
        for j, dref in enumerate((d0, d1, d2, d3, d4)):
            ohd = (dref[...] == lax.broadcasted_iota(jnp.int32, (BLK, 38), 1))
            dora_emb = jnp.dot(ohd.astype(jnp.float32), dtab[...],
                               preferred_element_type=jnp.float32)
            lo = 48 + 64 * j
            acc += jnp.dot(dora_emb, wt[lo:lo + 64, :],
                           preferred_element_type=jnp.float32)
        acc += jnp.dot(h_emb, wt[368:384, :], preferred_element_type=jnp.float32)
        out_ref[...] = acc + ib[...]


def _build_combined(tab_pad, scores, oya1, dsplit, hrs, ln_g, ln_b,
                    wst, sb, oyat, dtab, hwt, hb, wt, ib):
    nb = B // BLK
    full = lambda i: (0, 0)
    batch = lambda i: (jnp.maximum(i - 1, 0), 0)
    return pl.pallas_call(
        _tc_body,
        grid=(nb + 1,),
        in_specs=[
            pl.BlockSpec((TAB_PAD, D), full),
            pl.BlockSpec((BLK, 4), batch),
            pl.BlockSpec((BLK, 1), batch),
            pl.BlockSpec((BLK, 1), batch),
            pl.BlockSpec((BLK, 1), batch),
            pl.BlockSpec((BLK, 1), batch),
            pl.BlockSpec((BLK, 1), batch),
            pl.BlockSpec((BLK, 1), batch),
            pl.BlockSpec((BLK, 2), batch),
            pl.BlockSpec((1, 4), full),
            pl.BlockSpec((1, 4), full),
            pl.BlockSpec((4, 32), full),
            pl.BlockSpec((1, 32), full),
            pl.BlockSpec((4, 16), full),
            pl.BlockSpec((38, 64), full),
            pl.BlockSpec((2, 16), full),
            pl.BlockSpec((1, 16), full),
            pl.BlockSpec((384, D), full),
            pl.BlockSpec((1, D), full),
        ],
        out_specs=pl.BlockSpec((BLK, D), lambda i: (i, 0)),
        out_shape=jax.ShapeDtypeStruct((TAB_PAD + B, D), jnp.float32),
    )(tab_pad, scores, oya1, *dsplit, hrs, ln_g, ln_b,
      wst, sb, oyat, dtab, hwt, hb, wt, ib)


def _sc_gather(comb, act3d):
    mesh = plsc.VectorSubcoreMesh(core_axis_name="c", subcore_axis_name="s",
                                  num_cores=2, num_subcores=16)

    @functools.partial(
        pl.kernel,
        out_type=jax.ShapeDtypeStruct((B, S, D), jnp.float32),
        mesh=mesh,
        scratch_types=[
            pltpu.VMEM((RPW, SPAD), jnp.int32),
            pltpu.VMEM((3, S, D), jnp.float32),
            pltpu.VMEM((3, 16, D), jnp.float32),
            pltpu.SemaphoreType.DMA,
            pltpu.SemaphoreType.DMA,
            pltpu.SemaphoreType.DMA,
            pltpu.SemaphoreType.DMA,
            pltpu.SemaphoreType.DMA,
            pltpu.SemaphoreType.DMA,
        ],
    )
    def k(comb_hbm, act_hbm, out_hbm, idx_v, bufs, stages,
          g0, g1, g2, s0, s1, s2):
        gsem = (g0, g1, g2)
        ssem = (s0, s1, s2)
        nc = 2
        wid = lax.axis_index("s") * nc + lax.axis_index("c")
        b0 = wid * RPW
        pltpu.sync_copy(act_hbm.at[wid], idx_v)

        def fix(r, _):
            bsrc = b0 + r + TAB_PAD
            for off in (0, 16, 32, 34):
                v = idx_v[r, pl.ds(off, 16)]
                idx_v[r, pl.ds(off, 16)] = jnp.where(v == SENTINEL, bsrc, v)
            return 0

        lax.fori_loop(0, RPW, fix, 0)

        def g(r, slot):
            pltpu.async_copy(comb_hbm.at[idx_v.at[r, pl.ds(0, 48)]],
                             bufs.at[slot, pl.ds(0, 48)], gsem[slot])
            pltpu.async_copy(comb_hbm.at[idx_v.at[r, pl.ds(34, 16)]],
                             stages.at[slot], gsem[slot])

        def wg(r, slot):
            pltpu.make_async_copy(
                comb_hbm.at[idx_v.at[r, pl.ds(0, 48)]],
                bufs.at[slot, pl.ds(0, 48)], gsem[slot]).wait()
            pltpu.make_async_copy(
                comb_hbm.at[idx_v.at[r, pl.ds(34, 16)]],
                stages.at[slot], gsem[slot]).wait()
            patch(slot)

        def patch(slot):
            def pbody(c, _):
                o = 16 * c
                bufs[slot, 48, pl.ds(o, 16)] = stages[slot, 14, pl.ds(o, 16)]
                bufs[slot, 49, pl.ds(o, 16)] = stages[slot, 15, pl.ds(o, 16)]
                return 0

            lax.fori_loop(0, D // 16, pbody, 0)

        def s(r, slot):
            pltpu.async_copy(bufs.at[slot], out_hbm.at[b0 + r], ssem[slot])

        def ws(r, slot):
            pltpu.make_async_copy(
                bufs.at[slot], out_hbm.at[b0 + r], ssem[slot]).wait()

        g(0, 0)
        g(1, 1)
        wg(0, 0); s(0, 0); g(2, 2)
        wg(1, 1); s(1, 1); ws(0, 0); g(3, 0)
        wg(2, 2); s(2, 2); ws(1, 1); g(4, 1)

        def body(p, _):
            for q in range(3):
                r = 3 * p + q
                wg(r, q)
                s(r, q)
                ws(r - 1, (q + 2) % 3)
                g(r + 2, (q + 2) % 3)
            return 0

        lax.fori_loop(1, (RPW - 2) // 3, body, 0)

        r = RPW - 2
        wg(r, r % 3); s(r, r % 3); ws(r - 1, (r - 1) % 3)
        r = RPW - 1
        wg(r, r % 3); s(r, r % 3); ws(r - 1, (r - 1) % 3)
        ws(RPW - 1, (RPW - 1) % 3)

    return k(comb, act3d)


def kernel(scores, oya, dora, honba_riichi_sticks, action, mask, action_table,
           info_W, info_b, ln_g, ln_b, scores_W, scores_b, oya_table,
           dora_table, hrs_W, hrs_b):
    del mask
    tab_pad = jnp.zeros((TAB_PAD, D), jnp.float32).at[:NTAB].set(action_table)
    oya1 = oya.astype(jnp.int32).reshape(B, 1)
    dora_i = dora.astype(jnp.int32)
    dsplit = [dora_i[:, j:j + 1] for j in range(5)]
    comb = _build_combined(
        tab_pad, scores, oya1, dsplit, honba_riichi_sticks,
        ln_g.reshape(1, 4), ln_b.reshape(1, 4),
        scores_W.T, scores_b.reshape(1, 32),
        oya_table, dora_table,
        hrs_W.T, hrs_b.reshape(1, 16),
        info_W.T, info_b.reshape(1, D))
    act_pad = jnp.pad(action.astype(jnp.int32), ((0, 0), (0, SPAD - S)))
    act3d = act_pad.reshape(NW, RPW, SPAD)
    return _sc_gather(comb, act3d)


# speedup vs baseline: 1.1029x; 1.1029x over previous
import functools

import jax
import jax.numpy as jnp
from jax import lax
from jax.experimental import pallas as pl
from jax.experimental.pallas import tpu as pltpu
from jax.experimental.pallas import tpu_sc as plsc

B = 4096
S = 50
D = 512
NTAB = 225
TAB_PAD = 256
SENTINEL = 224

BLK = 256
NW = 32
RPW = B // NW
SPAD = 64


def _tc_body(tab_ref, sc_ref, oy_ref, d0, d1, d2, d3, d4, hr_ref,
             lng, lnb, wst, sb, oyat, dtab, hwt, hb, wt, ib, out_ref):
    i = pl.program_id(0)

    @pl.when(i == 0)
    def _():
        out_ref[...] = tab_ref[...]

    @pl.when(i > 0)
    def _():
        x = sc_ref[...]
        mu = jnp.mean(x, axis=-1, keepdims=True)
        xc = x - mu
        var = jnp.mean(xc * xc, axis=-1, keepdims=True)
        xn = xc * lax.rsqrt(var + 1e-5) * lng[...] + lnb[...]
        s_emb = jnp.dot(xn, wst[...], preferred_element_type=jnp.float32) + sb[...]

        oh = (oy_ref[...] == lax.broadcasted_iota(jnp.int32, (BLK, 4), 1))
        oya_emb = jnp.dot(oh.astype(jnp.float32), oyat[...],
                          preferred_element_type=jnp.float32)

        h_emb = jnp.dot(hr_ref[...], hwt[...],
                        preferred_element_type=jnp.float32) + hb[...]

        acc = jnp.dot(s_emb, wt[0:32, :], preferred_element_type=jnp.float32)
        acc += jnp.dot(oya_emb, wt[32:48, :], preferred_element_type=jnp.float32)
        for j, dref in enumerate((d0, d1, d2, d3, d4)):
            ohd = (dref[...] == lax.broadcasted_iota(jnp.int32, (BLK, 38), 1))
            dora_emb = jnp.dot(ohd.astype(jnp.float32), dtab[...],
                               preferred_element_type=jnp.float32)
            lo = 48 + 64 * j
            acc += jnp.dot(dora_emb, wt[lo:lo + 64, :],
                           preferred_element_type=jnp.float32)
        acc += jnp.dot(h_emb, wt[368:384, :], preferred_element_type=jnp.float32)
        out_ref[...] = acc + ib[...]


def _build_combined(tab_pad, scores, oya1, dsplit, hrs, ln_g, ln_b,
                    wst, sb, oyat, dtab, hwt, hb, wt, ib):
    nb = B // BLK
    full = lambda i: (0, 0)
    batch = lambda i: (jnp.maximum(i - 1, 0), 0)
    return pl.pallas_call(
        _tc_body,
        grid=(nb + 1,),
        in_specs=[
            pl.BlockSpec((TAB_PAD, D), full),
            pl.BlockSpec((BLK, 4), batch),
            pl.BlockSpec((BLK, 1), batch),
            pl.BlockSpec((BLK, 1), batch),
            pl.BlockSpec((BLK, 1), batch),
            pl.BlockSpec((BLK, 1), batch),
            pl.BlockSpec((BLK, 1), batch),
            pl.BlockSpec((BLK, 1), batch),
            pl.BlockSpec((BLK, 2), batch),
            pl.BlockSpec((1, 4), full),
            pl.BlockSpec((1, 4), full),
            pl.BlockSpec((4, 32), full),
            pl.BlockSpec((1, 32), full),
            pl.BlockSpec((4, 16), full),
            pl.BlockSpec((38, 64), full),
            pl.BlockSpec((2, 16), full),
            pl.BlockSpec((1, 16), full),
            pl.BlockSpec((384, D), full),
            pl.BlockSpec((1, D), full),
        ],
        out_specs=pl.BlockSpec((BLK, D), lambda i: (i, 0)),
        out_shape=jax.ShapeDtypeStruct((TAB_PAD + B, D), jnp.float32),
    )(tab_pad, scores, oya1, *dsplit, hrs, ln_g, ln_b,
      wst, sb, oyat, dtab, hwt, hb, wt, ib)


def _sc_gather(comb, act3d, tail3d):
    mesh = plsc.VectorSubcoreMesh(core_axis_name="c", subcore_axis_name="s",
                                  num_cores=2, num_subcores=16)

    @functools.partial(
        pl.kernel,
        out_type=jax.ShapeDtypeStruct((B, S, D), jnp.float32),
        mesh=mesh,
        scratch_types=[
            pltpu.VMEM((RPW, SPAD), jnp.int32),
            pltpu.VMEM((3, S, D), jnp.float32),
            pltpu.VMEM((2, 16, D), jnp.float32),
            pltpu.VMEM((16, 64), jnp.int32),
            pltpu.SemaphoreType.DMA,
            pltpu.SemaphoreType.DMA,
            pltpu.SemaphoreType.DMA,
            pltpu.SemaphoreType.DMA,
            pltpu.SemaphoreType.DMA,
            pltpu.SemaphoreType.DMA,
            pltpu.SemaphoreType.DMA,
            pltpu.SemaphoreType.DMA,
        ],
    )
    def k(comb_hbm, act_hbm, tail_hbm, out_hbm, idx_v, bufs, gstage, tidx,
          g0, g1, g2, s0, s1, s2, t0, t1):
        gsem = (g0, g1, g2)
        ssem = (s0, s1, s2)
        tsem = (t0, t1)
        nc = 2
        wid = lax.axis_index("s") * nc + lax.axis_index("c")
        b0 = wid * RPW
        pltpu.sync_copy(act_hbm.at[wid], idx_v)

        lane = lax.iota(jnp.int32, 16)

        def fix(r, _):
            bsrc = b0 + r + TAB_PAD
            for off in (0, 16, 32, 34):
                v = idx_v[r, pl.ds(off, 16)]
                idx_v[r, pl.ds(off, 16)] = jnp.where(v == SENTINEL, bsrc, v)
            return 0

        lax.fori_loop(0, RPW, fix, 0)

        pltpu.sync_copy(tail_hbm.at[wid], tidx)

        def tfix(gj, _):
            v = tidx[gj, pl.ds(0, 16)]
            bsrc = b0 + 8 * gj + (lane >> 1) + TAB_PAD
            tidx[gj, pl.ds(0, 16)] = jnp.where(v == SENTINEL, bsrc, v)
            return 0

        lax.fori_loop(0, RPW // 8, tfix, 0)

        def gt(gi, gp):
            pltpu.async_copy(comb_hbm.at[tidx.at[gi, pl.ds(0, 16)]], gstage.at[gp], tsem[gp])

        def wt(gi, gp):
            pltpu.make_async_copy(
                comb_hbm.at[tidx.at[gi, pl.ds(0, 16)]], gstage.at[gp], tsem[gp]).wait()

        def g(r, slot):
            pltpu.async_copy(comb_hbm.at[idx_v.at[r, pl.ds(0, 48)]],
                             bufs.at[slot, pl.ds(0, 48)], gsem[slot])

        def wg(r, slot, gp, j):
            pltpu.make_async_copy(
                comb_hbm.at[idx_v.at[r, pl.ds(0, 48)]],
                bufs.at[slot, pl.ds(0, 48)], gsem[slot]).wait()

            def pbody(c, _):
                o = 16 * c
                bufs[slot, 48, pl.ds(o, 16)] = gstage[gp, 2 * j, pl.ds(o, 16)]
                bufs[slot, 49, pl.ds(o, 16)] = gstage[gp, 2 * j + 1, pl.ds(o, 16)]
                return 0

            lax.fori_loop(0, D // 16, pbody, 0)

        def s(r, slot):
            pltpu.async_copy(bufs.at[slot], out_hbm.at[b0 + r], ssem[slot])

        def ws(r, slot):
            pltpu.make_async_copy(
                bufs.at[slot], out_hbm.at[b0 + r], ssem[slot]).wait()

        gt(0, 0)
        gt(1, 1)
        g(0, 0)
        g(1, 1)
        wt(0, 0)
        wg(0, 0, 0, 0); s(0, 0); g(2, 2)
        wg(1, 1, 0, 1); s(1, 1); ws(0, 0); g(3, 0)
        for j in range(2, 8):
            r = j
            wg(r, r % 3, 0, j); s(r, r % 3); ws(r - 1, (r - 1) % 3)
            g(r + 2, (r + 2) % 3)
        gt(2, 0)

        def group(gi, base3, gp, fire_next):
            wt(gi, gp)
            for j in range(8):
                r = 8 * gi + j
                wg(r, (base3 + j) % 3, gp, j)
                s(r, (base3 + j) % 3)
                ws(r - 1, (base3 + j - 1) % 3)
                if not isinstance(r, int) or r + 2 < RPW:
                    g(r + 2, (base3 + j + 2) % 3)
            if fire_next:
                gt(gi + 2, gp)

        def body(mm, _):
            for G in range(1, 7):
                group(6 * mm + G, (8 * G) % 3, G & 1, True)
            return 0

        lax.fori_loop(0, 2, body, 0)

        group(13, (8 * 13) % 3, 1, True)
        group(14, (8 * 14) % 3, 0, False)
        group(15, (8 * 15) % 3, 1, False)
        ws(RPW - 1, (RPW - 1) % 3)

    return k(comb, act3d, tail3d)


def kernel(scores, oya, dora, honba_riichi_sticks, action, mask, action_table,
           info_W, info_b, ln_g, ln_b, scores_W, scores_b, oya_table,
           dora_table, hrs_W, hrs_b):
    del mask
    tab_pad = jnp.zeros((TAB_PAD, D), jnp.float32).at[:NTAB].set(action_table)
    oya1 = oya.astype(jnp.int32).reshape(B, 1)
    dora_i = dora.astype(jnp.int32)
    dsplit = [dora_i[:, j:j + 1] for j in range(5)]
    comb = _build_combined(
        tab_pad, scores, oya1, dsplit, honba_riichi_sticks,
        ln_g.reshape(1, 4), ln_b.reshape(1, 4),
        scores_W.T, scores_b.reshape(1, 32),
        oya_table, dora_table,
        hrs_W.T, hrs_b.reshape(1, 16),
        info_W.T, info_b.reshape(1, D))
    act_i = action.astype(jnp.int32)
    act_pad = jnp.pad(act_i, ((0, 0), (0, SPAD - S)))
    act3d = act_pad.reshape(NW, RPW, SPAD)
    tail = act_i[:, 48:50].reshape(NW, RPW // 8, 16)
    tail3d = jnp.pad(tail, ((0, 0), (0, 0), (0, 48)))
    return _sc_gather(comb, act3d, tail3d)
